# baseline (speedup 1.0000x reference)
import functools

import jax
import jax.numpy as jnp
from jax import lax
from jax.experimental import pallas as pl
from jax.experimental.pallas import tpu as pltpu
from jax.experimental.pallas import tpu_sc as plsc

D_MODEL = 64


@functools.cache
def _make_gather(B: int, V: int, D: int):
    info = plsc.get_sparse_core_info()
    NC, NS = info.num_cores, info.num_subcores
    NW = NC * NS
    assert B % NW == 0
    b_per_w = B // NW
    C = 400
    NBUF = 4
    LAG = 2
    assert b_per_w % C == 0 and C % 8 == 0
    n_chunks = b_per_w // C
    assert n_chunks % NBUF == 0 and n_chunks > NBUF

    mesh = plsc.VectorSubcoreMesh(core_axis_name="c", subcore_axis_name="s")

    @functools.partial(
        pl.kernel,
        mesh=mesh,
        out_type=jax.ShapeDtypeStruct((B, D), jnp.float32),
        scratch_types=[
            pltpu.VMEM((b_per_w,), jnp.int32),
            pltpu.VMEM((NBUF, C, D), jnp.float32),
            pltpu.SemaphoreType.DMA,
            pltpu.SemaphoreType.DMA((NBUF,)),
            pltpu.SemaphoreType.DMA((NBUF,)),
        ],
        compiler_params=pltpu.CompilerParams(use_tc_tiling_on_sc=False),
    )
    def gather_kernel(idx_hbm, table_hbm, out_hbm, idx_v, rows_v, sem_i, sem_g, sem_o):
        wid = lax.axis_index("s") * NC + lax.axis_index("c")
        base = wid * b_per_w

        pltpu.async_copy(idx_hbm.at[pl.ds(base, b_per_w)], idx_v, sem_i).wait()

        def start_gather(i, b):
            pltpu.async_copy(
                table_hbm.at[idx_v.at[pl.ds(i * C, C)]], rows_v.at[b], sem_g.at[b]
            )

        def finish_gather_start_writeback(i, b):
            pltpu.make_async_copy(
                table_hbm.at[idx_v.at[pl.ds(i * C, C)]], rows_v.at[b], sem_g.at[b]
            ).wait()
            pltpu.async_copy(
                rows_v.at[b], out_hbm.at[pl.ds(base + i * C, C)], sem_o.at[b]
            )

        @pl.loop(0, n_chunks, step=NBUF)
        def _(g):
            for b in range(NBUF):
                i = g + b

                @pl.when(i >= NBUF)
                def _():
                    pltpu.make_async_copy(
                        rows_v.at[b], out_hbm.at[pl.ds(base, C)], sem_o.at[b]
                    ).wait()

                start_gather(i, b)

                @pl.when(i >= LAG)
                def _():
                    finish_gather_start_writeback(i - LAG, (b - LAG) % NBUF)

        for j in range(LAG):
            i = n_chunks - LAG + j
            finish_gather_start_writeback(i, i % NBUF)
        for b in range(NBUF):
            pltpu.make_async_copy(
                rows_v.at[b], out_hbm.at[pl.ds(base, C)], sem_o.at[b]
            ).wait()

    return gather_kernel


def kernel(x, table):
    B = x.shape[0] * x.shape[1]
    out = _make_gather(B, table.shape[0], D_MODEL)(x.reshape(B), table)
    return out.reshape(x.shape[0], x.shape[1], D_MODEL)


# speedup vs baseline: 1.3270x; 1.3270x over previous
import functools

import jax
import jax.numpy as jnp
from jax import lax
from jax.experimental import pallas as pl
from jax.experimental.pallas import tpu as pltpu
from jax.experimental.pallas import tpu_sc as plsc

D_MODEL = 64


@functools.cache
def _make_gather(B: int, S: int, V: int, D: int):
    info = plsc.get_sparse_core_info()
    NC, NS = info.num_cores, info.num_subcores
    NW = NC * NS
    assert B % NW == 0
    rows_per_w = B // NW
    NBUF = 4
    LAG = 2
    assert rows_per_w % NBUF == 0 and rows_per_w > NBUF

    mesh = plsc.VectorSubcoreMesh(core_axis_name="c", subcore_axis_name="s")

    @functools.partial(
        pl.kernel,
        mesh=mesh,
        out_type=jax.ShapeDtypeStruct((B, S, 2 * D), jnp.float32),
        scratch_types=[
            pltpu.VMEM((rows_per_w, S), jnp.int32),
            pltpu.VMEM((NBUF, S, D), jnp.float32),
            pltpu.SemaphoreType.DMA,
            pltpu.SemaphoreType.DMA((NBUF,)),
            pltpu.SemaphoreType.DMA((NBUF,)),
        ],
        compiler_params=pltpu.CompilerParams(use_tc_tiling_on_sc=False),
    )
    def gather_kernel(x_hbm, table_hbm, out_hbm, idx_v, rows_v, sem_i, sem_g, sem_o):
        wid = lax.axis_index("s") * NC + lax.axis_index("c")
        base = wid * rows_per_w

        pltpu.async_copy(x_hbm.at[pl.ds(base, rows_per_w)], idx_v, sem_i).wait()

        def start_gather(i, b):
            pltpu.async_copy(
                table_hbm.at[idx_v.at[i]], rows_v.at[b], sem_g.at[b]
            )

        def finish_gather_start_writeback(i, b):
            pltpu.make_async_copy(
                table_hbm.at[idx_v.at[i]], rows_v.at[b], sem_g.at[b]
            ).wait()
            pltpu.async_copy(
                rows_v.at[b], out_hbm.at[base + i, :, pl.ds(0, D)], sem_o.at[b]
            )

        @pl.loop(0, rows_per_w, step=NBUF)
        def _(g):
            for b in range(NBUF):
                i = g + b

                @pl.when(i >= NBUF)
                def _():
                    pltpu.make_async_copy(
                        rows_v.at[b], out_hbm.at[base, :, pl.ds(0, D)], sem_o.at[b]
                    ).wait()

                start_gather(i, b)

                @pl.when(i >= LAG)
                def _():
                    finish_gather_start_writeback(i - LAG, (b - LAG) % NBUF)

        for j in range(LAG):
            i = rows_per_w - LAG + j
            finish_gather_start_writeback(i, i % NBUF)
        for b in range(NBUF):
            pltpu.make_async_copy(
                rows_v.at[b], out_hbm.at[base, :, pl.ds(0, D)], sem_o.at[b]
            ).wait()

    return gather_kernel


def kernel(x, table):
    B, S = x.shape
    out = _make_gather(B, S, table.shape[0], D_MODEL)(x, table)
    return out[:, :, :D_MODEL]
